# Initial kernel scaffold; baseline (speedup 1.0000x reference)
import jax
import jax.numpy as jnp
from jax.experimental import pallas as pl


def kernel(eigenvalues, eigenvectors_sq, W1, b1, W2, b2, W3, b3, Wp, bp, birth_idx, death_idx):
    raise NotImplementedError("write your pallas kernel here")


# speedup vs baseline: 1.6938x; 1.6938x over previous
import functools

import jax
import jax.numpy as jnp
from jax.experimental import pallas as pl
from jax.experimental.pallas import tpu as pltpu
from jax.experimental.pallas import tpu_sc as plsc

_L, _N, _M, _K = 32, 1024, 128, 25
_PLANES = 3
_NC, _NS, _LANES = 2, 16, 16


def _r(x):
    u = jax.lax.bitcast_convert_type(x, jnp.uint32)
    r = (u + jnp.uint32(0x7FFF) + ((u >> 16) & jnp.uint32(1))) \
        & jnp.uint32(0xFFFF0000)
    return jax.lax.bitcast_convert_type(r, jnp.float32)


def _tc_body(ev_ref, w1_ref, b1_ref, w2_ref, b2_ref, w3_ref, b3_ref, e_ref,
             f_ref):
    l = pl.program_id(0)
    e = ev_ref[pl.ds(l, 1), :]
    h1 = [jnp.maximum(e * w1_ref[p, 0] + b1_ref[p], 0.0) for p in range(3)]
    h1 = [_r(h) for h in h1]
    h2 = []
    for g in range(5):
        t = (h1[0] * w2_ref[g, 0] + h1[1] * w2_ref[g, 1]
             + h1[2] * w2_ref[g, 2] + b2_ref[g])
        h2.append(jnp.maximum(t, 0.0))
    h2 = [_r(h) for h in h2]
    gv = (h2[0] * w3_ref[0, 0] + h2[1] * w3_ref[0, 1] + h2[2] * w3_ref[0, 2]
          + h2[3] * w3_ref[0, 3] + h2[4] * w3_ref[0, 4] + b3_ref[0])
    mu = jnp.mean(gv)
    dev = gv - mu
    var = jnp.mean(dev * dev)
    gn = dev / jnp.sqrt(var + 1e-5)
    emat = e_ref[0]
    f_ref[pl.ds(l, 1), :] = jax.lax.dot_general(
        gn.astype(jnp.bfloat16), emat.astype(jnp.bfloat16),
        (((1,), (1,)), ((), ())),
        preferred_element_type=jnp.float32)


def _tc_f(ev2, w1, b1, w2, b2, w3, b3, evecs):
    smem = pl.BlockSpec(memory_space=pltpu.SMEM)
    return pl.pallas_call(
        _tc_body,
        grid=(_L,),
        in_specs=[
            pl.BlockSpec((_L, _N), lambda l: (0, 0)),
            smem, smem, smem, smem, smem, smem,
            pl.BlockSpec((1, _N, _N), lambda l: (l, 0, 0)),
        ],
        out_specs=pl.BlockSpec((_L, _N), lambda l: (0, 0)),
        out_shape=jax.ShapeDtypeStruct((_L, _N), jnp.float32),
    )(ev2, w1, b1, w2, b2, w3, b3, evecs)


def _sc_body(f_hbm, bi_hbm, di_hbm, wpb_hbm, out_hbm,
             f_v, bi_v, di_v, b_v, d_v, wp_v, o_v):
    c = jax.lax.axis_index("c")
    s = jax.lax.axis_index("s")
    wid = s * _NC + c
    pltpu.sync_copy(f_hbm.at[wid], f_v)
    pltpu.sync_copy(bi_hbm.at[wid], bi_v)
    pltpu.sync_copy(di_hbm.at[wid], di_v)
    pltpu.sync_copy(wpb_hbm, wp_v)

    wpvec = wp_v[...]
    lanes = jax.lax.iota(jnp.int32, _LANES)
    neg_inf = jnp.full((_LANES,), -jnp.inf, jnp.float32)
    nchunks = _M // _LANES
    acc = jnp.zeros((_LANES,), jnp.float32)

    for plane in range(_PLANES):
        p_regs = []
        for j in range(nchunks):
            off = plane * _M + j * _LANES
            bi = bi_v[pl.ds(off, _LANES)]
            di = di_v[pl.ds(off, _LANES)]
            bvals = plsc.load_gather(f_v, [bi])
            dvals = plsc.load_gather(f_v, [di])
            b_v[pl.ds(j * _LANES, _LANES)] = bvals
            d_v[pl.ds(j * _LANES, _LANES)] = dvals
            p_regs.append(jnp.abs(dvals - bvals))

        def step(k, carry):
            ps = list(carry[:nchunks])
            prev_b, prev_d, first_b, first_d, cross = carry[nchunks:]
            m = ps[0]
            for j in range(1, nchunks):
                m = jnp.maximum(m, ps[j])
            mb = jnp.broadcast_to(jnp.max(m), (_LANES,))
            selv = jnp.full((_LANES,), 16384, jnp.int32)
            for j in range(nchunks):
                cand = jnp.where(ps[j] == mb, lanes + (j * _LANES), selv)
                selv = jnp.minimum(selv, cand)
            selb = jnp.broadcast_to(jnp.min(selv), (_LANES,))
            bs = plsc.load_gather(b_v, [selb])
            ds_ = plsc.load_gather(d_v, [selb])
            for j in range(nchunks):
                hit = (lanes + (j * _LANES)) == selb
                ps[j] = jnp.where(hit, neg_inf, ps[j])
            cross = cross + bs * prev_d - ds_ * prev_b
            is0 = jnp.broadcast_to(k == 0, (_LANES,))
            first_b = jnp.where(is0, bs, first_b)
            first_d = jnp.where(is0, ds_, first_d)
            return tuple(ps) + (bs, ds_, first_b, first_d, cross)

        z = jnp.zeros((_LANES,), jnp.float32)
        fin = jax.lax.fori_loop(0, _K, step, tuple(p_regs) + (z, z, z, z, z))
        last_b, last_d, first_b, first_d, cross = fin[nchunks:]
        incx = first_b - last_b
        incy = first_d - last_d
        area = 0.5 * (cross - (last_b * first_d - last_d * first_b))
        acc = (acc + incx * wpvec[3 * plane] + incy * wpvec[3 * plane + 1]
               + area * wpvec[3 * plane + 2])

    o_v[...] = acc + wpvec[3 * _PLANES]
    pltpu.sync_copy(o_v, out_hbm.at[wid])


def _sc_call(f, bidx2, didx2, wpb):
    mesh = plsc.VectorSubcoreMesh(core_axis_name="c", subcore_axis_name="s")
    run = functools.partial(
        pl.kernel,
        out_type=jax.ShapeDtypeStruct((_L, _LANES), jnp.float32),
        mesh=mesh,
        compiler_params=pltpu.CompilerParams(needs_layout_passes=False),
        scratch_types=[
            pltpu.VMEM((_N,), jnp.float32),
            pltpu.VMEM((_PLANES * _M,), jnp.int32),
            pltpu.VMEM((_PLANES * _M,), jnp.int32),
            pltpu.VMEM((_M,), jnp.float32),
            pltpu.VMEM((_M,), jnp.float32),
            pltpu.VMEM((_LANES,), jnp.float32),
            pltpu.VMEM((_LANES,), jnp.float32),
        ],
    )(_sc_body)
    return run(f, bidx2, didx2, wpb)


def kernel(eigenvalues, eigenvectors_sq, W1, b1, W2, b2, W3, b3, Wp, bp,
           birth_idx, death_idx):
    ev2 = eigenvalues.reshape(_L, _N)
    w1r = W1
    w2r = _r(W2)
    w3r = _r(W3)
    f = _tc_f(ev2, w1r, b1, w2r, b2, w3r, b3, eigenvectors_sq)
    bidx2 = birth_idx.reshape(_L, _PLANES * _M)
    didx2 = death_idx.reshape(_L, _PLANES * _M)
    wpb = jnp.concatenate([Wp.reshape(-1), bp.reshape(-1),
                           jnp.zeros(_LANES - 3 * _PLANES - 1, jnp.float32)])
    o = _sc_call(f, bidx2, didx2, wpb)
    return o[:, :1]
